# baseline (speedup 1.0000x reference)
import functools

import jax
import jax.numpy as jnp
from jax import lax
from jax.experimental import pallas as pl
from jax.experimental.pallas import tpu as pltpu
from jax.experimental.pallas import tpu_sc as plsc

_L = 16


def _ceil_to(x, m):
    return -(-x // m) * m


@functools.lru_cache(maxsize=None)
def _build(M, K, V):
    n_out = M * V
    n_pad = _ceil_to(n_out, _L)
    wout_off = M * K * V
    n_data = _ceil_to(wout_off + n_out, _L)
    n_idx = _ceil_to(2 * V, _L)
    scale = 1.0 / M
    mesh = plsc.ScalarSubcoreMesh(axis_name="c", num_cores=1)

    @functools.partial(
        pl.kernel,
        mesh=mesh,
        compiler_params=pltpu.CompilerParams(
            needs_layout_passes=False,
            disable_bounds_checks=True,
            disable_semaphore_checks=True,
            skip_device_barrier=True,
        ),
        out_type=jax.ShapeDtypeStruct((n_pad,), jnp.float32),
        scratch_types=[
            pltpu.SMEM((n_data,), jnp.float32),
            pltpu.SMEM((n_idx,), jnp.int32),
            pltpu.SMEM((n_pad,), jnp.float32),
        ],
    )
    def scs_kernel(data_hbm, idx_hbm, out_hbm, data_s, idx_s, out_s):
        pltpu.sync_copy(data_hbm, data_s)
        pltpu.sync_copy(idx_hbm, idx_s)
        for i in range(n_out):
            m, v = divmod(i, V)
            base = m * (K * V) + v
            i0 = idx_s[v]
            i1 = idx_s[v + V]
            a = data_s[base + i0 * V]
            b = data_s[base + i1 * V]
            w = data_s[wout_off + i]
            out_s[i] = scale * a * b * w
        pltpu.sync_copy(out_s, out_hbm)

    return scs_kernel


def kernel(num_M, num_VN, IVF, VN_index, wout):
    M, K, V = IVF.shape
    n_out = M * V
    wout_off = M * K * V
    n_data = _ceil_to(wout_off + n_out, _L)
    n_idx = _ceil_to(2 * V, _L)
    data = jnp.zeros((n_data,), jnp.float32)
    data = data.at[:wout_off].set(IVF.reshape(wout_off).astype(jnp.float32))
    data = data.at[wout_off : wout_off + n_out].set(
        wout.reshape(n_out).astype(jnp.float32)
    )
    idx = (
        jnp.zeros((n_idx,), jnp.int32)
        .at[: 2 * V]
        .set(VN_index.astype(jnp.int32).reshape(2 * V))
    )
    out = _build(M, K, V)(data, idx)
    return out[:n_out].reshape(M, V)


# speedup vs baseline: 1.1174x; 1.1174x over previous
import functools

import jax
import jax.numpy as jnp
from jax import lax
from jax.experimental import pallas as pl
from jax.experimental.pallas import tpu as pltpu
from jax.experimental.pallas import tpu_sc as plsc


@functools.lru_cache(maxsize=None)
def _build(M, K, V):
    scale = 1.0 / M
    mesh = plsc.ScalarSubcoreMesh(axis_name="c", num_cores=1)

    @functools.partial(
        pl.kernel,
        mesh=mesh,
        compiler_params=pltpu.CompilerParams(
            needs_layout_passes=False,
            disable_bounds_checks=True,
            disable_semaphore_checks=True,
            skip_device_barrier=True,
        ),
        out_type=jax.ShapeDtypeStruct((M, V), jnp.float32),
        scratch_types=[
            pltpu.SMEM((M, K, V), jnp.float32),
            pltpu.SMEM((2, V), jnp.int32),
            pltpu.SMEM((M, V), jnp.float32),
            pltpu.SMEM((M, V), jnp.float32),
            pltpu.SemaphoreType.DMA,
        ],
    )
    def scs_kernel(ivf_hbm, idx_hbm, wout_hbm, out_hbm, ivf_s, idx_s, wout_s, out_s, sem):
        c1 = pltpu.make_async_copy(ivf_hbm, ivf_s, sem)
        c2 = pltpu.make_async_copy(idx_hbm, idx_s, sem)
        c3 = pltpu.make_async_copy(wout_hbm, wout_s, sem)
        c1.start()
        c2.start()
        c3.start()
        c1.wait()
        c2.wait()
        c3.wait()
        for v in range(V):
            i0 = idx_s[0, v]
            i1 = idx_s[1, v]
            for m in range(M):
                a = ivf_s[m, i0, v]
                b = ivf_s[m, i1, v]
                out_s[m, v] = scale * a * b * wout_s[m, v]
        pltpu.sync_copy(out_s, out_hbm)

    return scs_kernel


def kernel(num_M, num_VN, IVF, VN_index, wout):
    M, K, V = IVF.shape
    return _build(M, K, V)(
        IVF.astype(jnp.float32),
        VN_index.astype(jnp.int32),
        wout.astype(jnp.float32),
    )
